# Initial kernel scaffold; baseline (speedup 1.0000x reference)
import jax
import jax.numpy as jnp
from jax.experimental import pallas as pl


def kernel(adj, edges_features, nodes_features, W_mu, b_mu, W_ls, b_ls, W_d1, b_d1, W_d2, b_d2, W_nd, b_nd, W_ed, b_ed, eps):
    raise NotImplementedError("write your pallas kernel here")


# speedup vs baseline: 232.8042x; 232.8042x over previous
import itertools
import functools

import numpy as np
import jax
import jax.numpy as jnp
from jax import lax
from jax.experimental import pallas as pl
from jax.experimental.pallas import tpu as pltpu
from jax.experimental.pallas import tpu_sc as plsc

_N = 9
_NSQ = _N * _N
_NPERM = 362880
_LANES = 16
_NW = 32
_PER_W_VECS = 720
_PER_W = _PER_W_VECS * _LANES
_NPAD = _NW * _PER_W

_PERMS_NP = np.array(list(itertools.permutations(range(_N))), dtype=np.int32)
_PERMS_PAD = np.concatenate(
    [_PERMS_NP, np.broadcast_to(_PERMS_NP[0], (_NPAD - _NPERM, _N)).copy()], axis=0)
_IDXT_NP = (np.arange(_N, dtype=np.int32)[None, :, None] * _N
            + _PERMS_PAD.reshape(_NW, _PER_W, _N).transpose(0, 2, 1)
            ).reshape(_NW, _N * _PER_W).copy()
_FACT = [1, 1, 2, 6, 24, 120, 720, 5040, 40320]

_TRIU, _TRIV = np.triu_indices(_N)
_T1U, _T1V = np.triu_indices(_N, 1)
_NTRI = _TRIU.shape[0]

_T2_NP = np.zeros((_NTRI, _NSQ), np.float32)
for _e in range(_NTRI):
    _u, _v = int(_TRIU[_e]), int(_TRIV[_e])
    _T2_NP[_e, _u * _N + _v] = 1.0
    _T2_NP[_e, _v * _N + _u] = 1.0

_TRI1_NP = np.zeros((_NSQ, 36), np.float32)
for _e in range(36):
    _TRI1_NP[int(_T1U[_e]) * _N + int(_T1V[_e]), _e] = 1.0

_DSEL_NP = np.zeros((_N, _NSQ), np.float32)
for _i in range(_N):
    _DSEL_NP[_i, _i * _N + _i] = 1.0
_DSELT_NP = _DSEL_NP.T.copy()

_BCOL_NP = np.zeros((_N, _NSQ), np.float32)
_BROW_NP = np.zeros((_NSQ, _N), np.float32)
_GSUM_NP = np.zeros((_N, _NSQ), np.float32)
for _a in range(_N):
    for _b in range(_N):
        _BCOL_NP[_b, _a * _N + _b] = 1.0
        _BROW_NP[_a * _N + _b, _b] = 1.0
        _GSUM_NP[_a, _a * _N + _b] = 1.0

_NOTI_NP = np.array([[0.0 if r // _N == r % _N else 1.0] for r in range(_NSQ)],
                    np.float32)
_OFFR_NP = _NOTI_NP.reshape(1, _NSQ).copy()

_UE_NP = np.zeros((48,), np.int32)
_VE_NP = np.zeros((48,), np.int32)
_UE_NP[:_NTRI] = _TRIU
_VE_NP[:_NTRI] = _TRIV


def _tc_body(gh, wmu, bmu, wls, bls, eps, wd1, bd1, wd2, bd2, wnd, bnd,
             wed4, bed4, nf, ef4, ef9, adja_row, adja_col,
             t2, tri1, dsel, dselt, bcol, brow, gsum, noti, offr,
             x_out, lp_out, l1p_out, part_out):
    f32 = jnp.float32
    dot = functools.partial(jnp.dot, preferred_element_type=f32,
                            precision=lax.Precision.HIGHEST)

    g = gh[...]
    zmu = dot(g, wmu[...]) + bmu[...]
    zls = dot(g, wls[...]) + bls[...]
    z = zmu + eps[...] * jnp.exp(0.5 * zls)
    y = jnp.maximum(dot(z, wd1[...]) + bd1[...], 0.0)
    out45 = jax.nn.sigmoid(dot(y, wd2[...]) + bd2[...])
    nr = dot(y, wnd[...]) + bnd[...]

    erows = [dot(y, wed4[f]) + bed4[f].reshape(1, 36) for f in range(4)]
    elin = jnp.concatenate(erows, axis=0)
    em = jnp.max(elin, axis=0, keepdims=True)
    ex = jnp.exp(elin - em)
    er = ex / jnp.sum(ex, axis=0, keepdims=True)

    er9 = er[:, :_N]
    e9 = ef9[...]
    dots = dot(e9, er9)
    nef = jnp.sqrt(jnp.sum(e9 * e9, axis=1, keepdims=True))
    nefr = jnp.sqrt(jnp.sum(er9 * er9, axis=0, keepdims=True))
    cosm = dots / jnp.maximum(nef * nefr, 1e-8)

    arow = adja_row[...]
    acol = adja_col[...]
    adjr = dot(out45, t2[...])
    diag_a = dot(dsel[...], acol)
    diag_r = dot(adjr, dselt[...])
    diag_term = diag_a * diag_r * cosm

    s_mat = jnp.abs(acol - adjr) * noti[...] * offr[...]
    s_mat = s_mat + dot(dselt[...], dot(diag_term, dsel[...]))

    bc = bcol[...]
    br = brow[...]
    gs = gsum[...]
    nc = noti[...]

    def mpm_body(_, x):
        xb = dot(br, dot(x, bc))
        tmp = s_mat * xb
        pm = jnp.concatenate(
            [jnp.max(tmp[:, 9 * c:9 * c + 9], axis=1, keepdims=True)
             for c in range(_N)], axis=1)
        neigh = dot(gs, pm * nc)
        xn = x * diag_term + neigh
        return xn / jnp.sqrt(jnp.sum(xn * xn))

    x = lax.fori_loop(0, 50, mpm_body, jnp.full((_N, _N), 1.0 / _N, f32))
    x_out[...] = x

    p = jnp.clip(out45, 1e-7, 1.0 - 1e-7)
    zpad = jnp.zeros((1, 3), f32)
    lp_out[...] = jnp.concatenate([jnp.log(p), zpad], axis=1)
    l1p_out[...] = jnp.concatenate([jnp.log(1.0 - p), zpad], axis=1)

    loss_kl = -0.5 * jnp.sum(1.0 + zls - zmu * zmu - jnp.exp(zls)) / float(_NSQ)
    loss_node = jnp.mean((nr - nf[...]) ** 2)
    aw = dot(arow, tri1[...])
    loss_edge = jnp.mean((er * aw - ef4[...]) ** 2)
    part = loss_kl + loss_node + loss_edge
    lane = lax.broadcasted_iota(jnp.int32, (1, 8), 1)
    part_out[...] = jnp.where(lane == 0, part, 0.0)


def _sc_body(idxt, a96, adj96, lp48, l1p48, part16, ue48, ve48,
             out_hbm,
             buf, av, sh_s, sh_i, tmpf, tmpi, availv, indv, adjv,
             lpv, l1pv, partv, uev, vev, outv):
    cid = lax.axis_index("c")
    sid = lax.axis_index("s")
    wid = sid * 2 + cid
    lanes = lax.iota(jnp.int32, _LANES)

    pltpu.sync_copy(a96, av)
    pltpu.sync_copy(idxt.at[wid], buf)

    base = wid * _PER_W

    def vec_body(v, carry):
        best, bidx = carry
        acc = jnp.zeros((_LANES,), jnp.float32)
        for k in range(_N):
            iv = buf[pl.ds(k * _PER_W + v * _LANES, _LANES)]
            acc = acc + plsc.load_gather(av, [iv])
        cur = base + v * _LANES + lanes
        upd = acc > best
        return jnp.where(upd, acc, best), jnp.where(upd, cur, bidx)

    best, bidx = lax.fori_loop(
        0, _PER_W_VECS, vec_body,
        (jnp.full((_LANES,), -jnp.inf, jnp.float32),
         jnp.zeros((_LANES,), jnp.int32)))

    tmpf[...] = best
    tmpi[...] = bidx
    pltpu.sync_copy(tmpf, sh_s.at[wid])
    pltpu.sync_copy(tmpi, sh_i.at[wid])
    plsc.subcore_barrier()

    @pl.when(wid == 0)
    def _():
        def red_body(t, carry):
            g, gi = carry
            pltpu.sync_copy(sh_s.at[t], tmpf)
            pltpu.sync_copy(sh_i.at[t], tmpi)
            s = tmpf[...]
            i = tmpi[...]
            better = (s > g) | ((s == g) & (i < gi))
            return jnp.where(better, s, g), jnp.where(better, i, gi)

        g, gi = lax.fori_loop(
            0, _NW, red_body,
            (jnp.full((_LANES,), -jnp.inf, jnp.float32),
             jnp.full((_LANES,), 2 ** 30, jnp.int32)))
        m = jnp.max(g)
        fi = jnp.min(jnp.where(g == m, gi, 2 ** 30))

        availv[...] = lanes
        rem = fi
        colv = jnp.zeros((_LANES,), jnp.int32)
        for k in range(_N):
            fct = _FACT[_N - 1 - k]
            d = rem // fct
            rem = rem - d * fct
            v = availv[...]
            elem = jnp.sum(jnp.where(lanes == d, v, 0))
            colv = jnp.where(lanes == k, elem, colv)
            shifted = plsc.load_gather(
                availv, [jnp.minimum(lanes + 1, _LANES - 1)])
            availv[...] = jnp.where(lanes >= d, shifted, v)
        indv[...] = jnp.zeros((_LANES,), jnp.int32)
        plsc.store_scatter(indv, [colv], lanes, mask=lanes < _N)

        pltpu.sync_copy(adj96, adjv)
        pltpu.sync_copy(lp48, lpv)
        pltpu.sync_copy(l1p48, l1pv)
        pltpu.sync_copy(part16, partv)
        pltpu.sync_copy(ue48, uev)
        pltpu.sync_copy(ve48, vev)

        acc = jnp.zeros((_LANES,), jnp.float32)
        for e3 in range(3):
            ue = uev[pl.ds(e3 * _LANES, _LANES)]
            ve = vev[pl.ds(e3 * _LANES, _LANES)]
            iu = plsc.load_gather(indv, [ue])
            iv2 = plsc.load_gather(indv, [ve])
            a_e = plsc.load_gather(adjv, [iu * _N + iv2])
            lpe = lpv[pl.ds(e3 * _LANES, _LANES)]
            l1pe = l1pv[pl.ds(e3 * _LANES, _LANES)]
            valid = (lanes + e3 * _LANES) < _NTRI
            acc = acc + jnp.where(valid, a_e * lpe + (1.0 - a_e) * l1pe, 0.0)
        total = jnp.sum(acc) * (-1.0 / float(_NTRI)) + jnp.sum(partv[...])
        outv[...] = lax.broadcast(total, (_LANES,))
        pltpu.sync_copy(outv, out_hbm)


def _run_tc(gh, wmu, bmu, wls, bls, eps, wd1, bd1, wd2, bd2, wnd, bnd,
            wed4, bed4, nf, ef4, ef9, adja_row, adja_col, interpret=False):
    f32 = jnp.float32
    consts = [jnp.asarray(c) for c in
              (_T2_NP, _TRI1_NP, _DSEL_NP, _DSELT_NP, _BCOL_NP, _BROW_NP,
               _GSUM_NP, _NOTI_NP, _OFFR_NP)]
    out_shapes = (jax.ShapeDtypeStruct((_N, _N), f32),
                  jax.ShapeDtypeStruct((1, 48), f32),
                  jax.ShapeDtypeStruct((1, 48), f32),
                  jax.ShapeDtypeStruct((1, 8), f32))
    return pl.pallas_call(_tc_body, out_shape=out_shapes, interpret=interpret)(
        gh, wmu, bmu, wls, bls, eps, wd1, bd1, wd2, bd2, wnd, bnd,
        wed4, bed4, nf, ef4, ef9, adja_row, adja_col, *consts)


def _run_sc(a96, adj96, lp48, l1p48, part16):
    i32, f32 = jnp.int32, jnp.float32
    mesh = plsc.VectorSubcoreMesh(core_axis_name="c", subcore_axis_name="s")
    scan = pl.kernel(
        _sc_body,
        out_type=jax.ShapeDtypeStruct((_LANES,), f32),
        mesh=mesh,
        compiler_params=pltpu.CompilerParams(needs_layout_passes=False),
        scratch_types=[
            pltpu.VMEM((_N * _PER_W,), i32),
            pltpu.VMEM((96,), f32),
            pltpu.VMEM_SHARED((_NW, _LANES), f32),
            pltpu.VMEM_SHARED((_NW, _LANES), i32),
            pltpu.VMEM((_LANES,), f32),
            pltpu.VMEM((_LANES,), i32),
            pltpu.VMEM((_LANES,), i32),
            pltpu.VMEM((_LANES,), i32),
            pltpu.VMEM((96,), f32),
            pltpu.VMEM((48,), f32),
            pltpu.VMEM((48,), f32),
            pltpu.VMEM((_LANES,), f32),
            pltpu.VMEM((48,), i32),
            pltpu.VMEM((48,), i32),
            pltpu.VMEM((_LANES,), f32),
        ])
    return scan(jnp.asarray(_IDXT_NP),
                a96, adj96, lp48, l1p48, part16,
                jnp.asarray(_UE_NP), jnp.asarray(_VE_NP))


def kernel(adj, edges_features, nodes_features, W_mu, b_mu, W_ls, b_ls,
           W_d1, b_d1, W_d2, b_d2, W_nd, b_nd, W_ed, b_ed, eps):
    f32 = jnp.float32
    gh = nodes_features.reshape(1, _N * 11).astype(f32)
    adj0 = adj[0].astype(f32)
    wed4 = W_ed.reshape(128, 36, 4).transpose(2, 0, 1)
    bed4 = b_ed.reshape(36, 4).T

    x, lp48, l1p48, part8 = _run_tc(
        gh, W_mu, b_mu.reshape(1, -1), W_ls, b_ls.reshape(1, -1), eps,
        W_d1, b_d1.reshape(1, -1), W_d2, b_d2.reshape(1, -1),
        W_nd, b_nd.reshape(1, -1), wed4, bed4,
        gh, edges_features[0].T, edges_features[0, :_N, :],
        adj0.reshape(1, _NSQ), adj0.reshape(_NSQ, 1))

    a96 = jnp.pad(x.reshape(_NSQ), (0, 96 - _NSQ))
    adj96 = jnp.pad(adj0.reshape(_NSQ), (0, 96 - _NSQ))
    part16 = jnp.pad(part8.reshape(8), (0, 8))
    res = _run_sc(a96, adj96, lp48.reshape(48), l1p48.reshape(48), part16)
    return res[0]


# speedup vs baseline: 233.0981x; 1.0013x over previous
import itertools
import functools

import numpy as np
import jax
import jax.numpy as jnp
from jax import lax
from jax.experimental import pallas as pl
from jax.experimental.pallas import tpu as pltpu
from jax.experimental.pallas import tpu_sc as plsc

_N = 9
_NSQ = _N * _N
_NPERM = 362880
_LANES = 16
_NW = 32
_PER_W_VECS = 720
_PER_W = _PER_W_VECS * _LANES
_NPAD = _NW * _PER_W

_PERMS_NP = np.array(list(itertools.permutations(range(_N))), dtype=np.int32)
_PERMS_PAD = np.concatenate(
    [_PERMS_NP, np.broadcast_to(_PERMS_NP[0], (_NPAD - _NPERM, _N)).copy()], axis=0)
_IDXT_NP = (np.arange(_N, dtype=np.int32)[None, :, None] * _N
            + _PERMS_PAD.reshape(_NW, _PER_W, _N).transpose(0, 2, 1)
            ).reshape(_NW, _N * _PER_W).copy()
_FACT = [1, 1, 2, 6, 24, 120, 720, 5040, 40320]

_TRIU, _TRIV = np.triu_indices(_N)
_T1U, _T1V = np.triu_indices(_N, 1)
_NTRI = _TRIU.shape[0]

_T2_NP = np.zeros((_NTRI, _NSQ), np.float32)
for _e in range(_NTRI):
    _u, _v = int(_TRIU[_e]), int(_TRIV[_e])
    _T2_NP[_e, _u * _N + _v] = 1.0
    _T2_NP[_e, _v * _N + _u] = 1.0

_TRI1_NP = np.zeros((_NSQ, 36), np.float32)
for _e in range(36):
    _TRI1_NP[int(_T1U[_e]) * _N + int(_T1V[_e]), _e] = 1.0

_DSEL_NP = np.zeros((_N, _NSQ), np.float32)
for _i in range(_N):
    _DSEL_NP[_i, _i * _N + _i] = 1.0
_DSELT_NP = _DSEL_NP.T.copy()

_BCOL_NP = np.zeros((_N, _NSQ), np.float32)
_BROW_NP = np.zeros((_NSQ, _N), np.float32)
_GSUM_NP = np.zeros((_N, _NSQ), np.float32)
for _a in range(_N):
    for _b in range(_N):
        _BCOL_NP[_b, _a * _N + _b] = 1.0
        _BROW_NP[_a * _N + _b, _b] = 1.0
        _GSUM_NP[_a, _a * _N + _b] = 1.0

_NOTI_NP = np.array([[0.0 if r // _N == r % _N else 1.0] for r in range(_NSQ)],
                    np.float32)
_OFFR_NP = _NOTI_NP.reshape(1, _NSQ).copy()

_UE_NP = np.zeros((48,), np.int32)
_VE_NP = np.zeros((48,), np.int32)
_UE_NP[:_NTRI] = _TRIU
_VE_NP[:_NTRI] = _TRIV


def _tc_body(gh, wmu, bmu, wls, bls, eps, wd1, bd1, wd2, bd2, wnd, bnd,
             wed4, bed4, nf, ef4, ef9, adja_row, adja_col,
             t2, tri1, dsel, dselt, bcol, brow, gsum, noti, offr,
             x_out, lp_out, l1p_out, part_out):
    f32 = jnp.float32
    bf16 = jnp.bfloat16
    dot = functools.partial(jnp.dot, preferred_element_type=f32,
                            precision=lax.Precision.HIGHEST)
    def bdot(a, b):
        ah = a.astype(bf16)
        al = (a - ah.astype(f32)).astype(bf16)
        bh = b.astype(bf16)
        d = lambda x, y: jnp.dot(x, y, preferred_element_type=f32)
        return d(ah, bh) + d(al, bh)

    g = gh[...]
    zmu = bdot(g, wmu[...]) + bmu[...]
    zls = bdot(g, wls[...]) + bls[...]
    z = zmu + eps[...] * jnp.exp(0.5 * zls)
    y = jnp.maximum(bdot(z, wd1[...]) + bd1[...], 0.0)
    out45 = jax.nn.sigmoid(bdot(y, wd2[...]) + bd2[...])
    nr = bdot(y, wnd[...]) + bnd[...]

    erows = [bdot(y, wed4[f]) + bed4[f].reshape(1, 36) for f in range(4)]
    elin = jnp.concatenate(erows, axis=0)
    em = jnp.max(elin, axis=0, keepdims=True)
    ex = jnp.exp(elin - em)
    er = ex / jnp.sum(ex, axis=0, keepdims=True)

    er9 = er[:, :_N]
    e9 = ef9[...]
    dots = dot(e9, er9)
    nef = jnp.sqrt(jnp.sum(e9 * e9, axis=1, keepdims=True))
    nefr = jnp.sqrt(jnp.sum(er9 * er9, axis=0, keepdims=True))
    cosm = dots / jnp.maximum(nef * nefr, 1e-8)

    arow = adja_row[...]
    acol = adja_col[...]
    adjr = dot(out45, t2[...])
    diag_a = dot(dsel[...], acol)
    diag_r = dot(adjr, dselt[...])
    diag_term = diag_a * diag_r * cosm

    s_mat = jnp.abs(acol - adjr) * noti[...] * offr[...]
    s_mat = s_mat + dot(dselt[...], dot(diag_term, dsel[...]))

    bc = bcol[...]
    br = brow[...]
    gs = gsum[...]
    nc = noti[...]

    def mpm_body(_, x):
        xb = dot(br, dot(x, bc))
        tmp = s_mat * xb
        pm = jnp.concatenate(
            [jnp.max(tmp[:, 9 * c:9 * c + 9], axis=1, keepdims=True)
             for c in range(_N)], axis=1)
        neigh = dot(gs, pm * nc)
        xn = x * diag_term + neigh
        return xn / jnp.sqrt(jnp.sum(xn * xn))

    x = lax.fori_loop(0, 50, mpm_body, jnp.full((_N, _N), 1.0 / _N, f32))
    x_out[...] = x

    p = jnp.clip(out45, 1e-7, 1.0 - 1e-7)
    zpad = jnp.zeros((1, 3), f32)
    lp_out[...] = jnp.concatenate([jnp.log(p), zpad], axis=1)
    l1p_out[...] = jnp.concatenate([jnp.log(1.0 - p), zpad], axis=1)

    loss_kl = -0.5 * jnp.sum(1.0 + zls - zmu * zmu - jnp.exp(zls)) / float(_NSQ)
    loss_node = jnp.mean((nr - nf[...]) ** 2)
    aw = dot(arow, tri1[...])
    loss_edge = jnp.mean((er * aw - ef4[...]) ** 2)
    part = loss_kl + loss_node + loss_edge
    lane = lax.broadcasted_iota(jnp.int32, (1, 8), 1)
    part_out[...] = jnp.where(lane == 0, part, 0.0)


def _sc_body(idxt, a96, adj96, lp48, l1p48, part16, ue48, ve48,
             out_hbm,
             buf, av, sh_s, sh_i, tmpf, tmpi, availv, indv, adjv,
             lpv, l1pv, partv, uev, vev, outv):
    cid = lax.axis_index("c")
    sid = lax.axis_index("s")
    wid = sid * 2 + cid
    lanes = lax.iota(jnp.int32, _LANES)

    pltpu.sync_copy(a96, av)
    pltpu.sync_copy(idxt.at[wid], buf)

    base = wid * _PER_W

    def vec_body(v, carry):
        best, bidx = carry
        acc = jnp.zeros((_LANES,), jnp.float32)
        for k in range(_N):
            iv = buf[pl.ds(k * _PER_W + v * _LANES, _LANES)]
            acc = acc + plsc.load_gather(av, [iv])
        cur = base + v * _LANES + lanes
        upd = acc > best
        return jnp.where(upd, acc, best), jnp.where(upd, cur, bidx)

    best, bidx = lax.fori_loop(
        0, _PER_W_VECS, vec_body,
        (jnp.full((_LANES,), -jnp.inf, jnp.float32),
         jnp.zeros((_LANES,), jnp.int32)))

    tmpf[...] = best
    tmpi[...] = bidx
    pltpu.sync_copy(tmpf, sh_s.at[wid])
    pltpu.sync_copy(tmpi, sh_i.at[wid])
    plsc.subcore_barrier()

    @pl.when(wid == 0)
    def _():
        def red_body(t, carry):
            g, gi = carry
            pltpu.sync_copy(sh_s.at[t], tmpf)
            pltpu.sync_copy(sh_i.at[t], tmpi)
            s = tmpf[...]
            i = tmpi[...]
            better = (s > g) | ((s == g) & (i < gi))
            return jnp.where(better, s, g), jnp.where(better, i, gi)

        g, gi = lax.fori_loop(
            0, _NW, red_body,
            (jnp.full((_LANES,), -jnp.inf, jnp.float32),
             jnp.full((_LANES,), 2 ** 30, jnp.int32)))
        m = jnp.max(g)
        fi = jnp.min(jnp.where(g == m, gi, 2 ** 30))

        availv[...] = lanes
        rem = fi
        colv = jnp.zeros((_LANES,), jnp.int32)
        for k in range(_N):
            fct = _FACT[_N - 1 - k]
            d = rem // fct
            rem = rem - d * fct
            v = availv[...]
            elem = jnp.sum(jnp.where(lanes == d, v, 0))
            colv = jnp.where(lanes == k, elem, colv)
            shifted = plsc.load_gather(
                availv, [jnp.minimum(lanes + 1, _LANES - 1)])
            availv[...] = jnp.where(lanes >= d, shifted, v)
        indv[...] = jnp.zeros((_LANES,), jnp.int32)
        plsc.store_scatter(indv, [colv], lanes, mask=lanes < _N)

        pltpu.sync_copy(adj96, adjv)
        pltpu.sync_copy(lp48, lpv)
        pltpu.sync_copy(l1p48, l1pv)
        pltpu.sync_copy(part16, partv)
        pltpu.sync_copy(ue48, uev)
        pltpu.sync_copy(ve48, vev)

        acc = jnp.zeros((_LANES,), jnp.float32)
        for e3 in range(3):
            ue = uev[pl.ds(e3 * _LANES, _LANES)]
            ve = vev[pl.ds(e3 * _LANES, _LANES)]
            iu = plsc.load_gather(indv, [ue])
            iv2 = plsc.load_gather(indv, [ve])
            a_e = plsc.load_gather(adjv, [iu * _N + iv2])
            lpe = lpv[pl.ds(e3 * _LANES, _LANES)]
            l1pe = l1pv[pl.ds(e3 * _LANES, _LANES)]
            valid = (lanes + e3 * _LANES) < _NTRI
            acc = acc + jnp.where(valid, a_e * lpe + (1.0 - a_e) * l1pe, 0.0)
        total = jnp.sum(acc) * (-1.0 / float(_NTRI)) + jnp.sum(partv[...])
        outv[...] = lax.broadcast(total, (_LANES,))
        pltpu.sync_copy(outv, out_hbm)


def _run_tc(gh, wmu, bmu, wls, bls, eps, wd1, bd1, wd2, bd2, wnd, bnd,
            wed4, bed4, nf, ef4, ef9, adja_row, adja_col, interpret=False):
    f32 = jnp.float32
    consts = [jnp.asarray(c) for c in
              (_T2_NP, _TRI1_NP, _DSEL_NP, _DSELT_NP, _BCOL_NP, _BROW_NP,
               _GSUM_NP, _NOTI_NP, _OFFR_NP)]
    out_shapes = (jax.ShapeDtypeStruct((_N, _N), f32),
                  jax.ShapeDtypeStruct((1, 48), f32),
                  jax.ShapeDtypeStruct((1, 48), f32),
                  jax.ShapeDtypeStruct((1, 8), f32))
    return pl.pallas_call(_tc_body, out_shape=out_shapes, interpret=interpret)(
        gh, wmu, bmu, wls, bls, eps, wd1, bd1, wd2, bd2, wnd, bnd,
        wed4, bed4, nf, ef4, ef9, adja_row, adja_col, *consts)


def _run_sc(a96, adj96, lp48, l1p48, part16):
    i32, f32 = jnp.int32, jnp.float32
    mesh = plsc.VectorSubcoreMesh(core_axis_name="c", subcore_axis_name="s")
    scan = pl.kernel(
        _sc_body,
        out_type=jax.ShapeDtypeStruct((_LANES,), f32),
        mesh=mesh,
        compiler_params=pltpu.CompilerParams(needs_layout_passes=False),
        scratch_types=[
            pltpu.VMEM((_N * _PER_W,), i32),
            pltpu.VMEM((96,), f32),
            pltpu.VMEM_SHARED((_NW, _LANES), f32),
            pltpu.VMEM_SHARED((_NW, _LANES), i32),
            pltpu.VMEM((_LANES,), f32),
            pltpu.VMEM((_LANES,), i32),
            pltpu.VMEM((_LANES,), i32),
            pltpu.VMEM((_LANES,), i32),
            pltpu.VMEM((96,), f32),
            pltpu.VMEM((48,), f32),
            pltpu.VMEM((48,), f32),
            pltpu.VMEM((_LANES,), f32),
            pltpu.VMEM((48,), i32),
            pltpu.VMEM((48,), i32),
            pltpu.VMEM((_LANES,), f32),
        ])
    return scan(jnp.asarray(_IDXT_NP),
                a96, adj96, lp48, l1p48, part16,
                jnp.asarray(_UE_NP), jnp.asarray(_VE_NP))


def kernel(adj, edges_features, nodes_features, W_mu, b_mu, W_ls, b_ls,
           W_d1, b_d1, W_d2, b_d2, W_nd, b_nd, W_ed, b_ed, eps):
    f32 = jnp.float32
    gh = nodes_features.reshape(1, _N * 11).astype(f32)
    adj0 = adj[0].astype(f32)
    wed4 = W_ed.reshape(128, 36, 4).transpose(2, 0, 1)
    bed4 = b_ed.reshape(36, 4).T

    x, lp48, l1p48, part8 = _run_tc(
        gh, W_mu, b_mu.reshape(1, -1), W_ls, b_ls.reshape(1, -1), eps,
        W_d1, b_d1.reshape(1, -1), W_d2, b_d2.reshape(1, -1),
        W_nd, b_nd.reshape(1, -1), wed4, bed4,
        gh, edges_features[0].T, edges_features[0, :_N, :],
        adj0.reshape(1, _NSQ), adj0.reshape(_NSQ, 1))

    a96 = jnp.pad(x.reshape(_NSQ), (0, 96 - _NSQ))
    adj96 = jnp.pad(adj0.reshape(_NSQ), (0, 96 - _NSQ))
    part16 = jnp.pad(part8.reshape(8), (0, 8))
    res = _run_sc(a96, adj96, lp48.reshape(48), l1p48.reshape(48), part16)
    return res[0]
